# Initial kernel scaffold; baseline (speedup 1.0000x reference)
import functools
import numpy as np
import jax
import jax.numpy as jnp
from jax import lax
from jax.experimental import pallas as pl
from jax.experimental.pallas import tpu as pltpu
from jax.experimental.pallas import tpu_sc as plsc

_D = 256
_H = 2
_L = 2
_P = 4
_SPATIAL = ((64, 64), (32, 32))
_BS = 2
_NQ = _SPATIAL[0][0] * _SPATIAL[0][1]
_NV = sum(h * w for h, w in _SPATIAL)
_NCOMBO = _H * _L * _P
_K = _L * _P * 4
_NOUT = _BS * _NQ * _H

_BQ = 512

_combo_l = np.array([(c % 8) // 4 for c in range(_NCOMBO)])
_W_L = np.array([_SPATIAL[l][1] for l in _combo_l], np.float32)
_H_L = np.array([_SPATIAL[l][0] for l in _combo_l], np.float32)
_START_L = np.array([0 if l == 0 else _SPATIAL[0][0] * _SPATIAL[0][1]
                     for l in _combo_l], np.int32)


def _prep_body(q_ref, ql_ref, wsox_ref, wsoy_ref, bsox_ref, bsoy_ref,
               waw_ref, baw_ref, idx_ref, wt_ref):
    b = pl.program_id(0) // (_NQ // _BQ)
    q = q_ref[...]
    dn = (((1,), (1,)), ((), ()))
    X = lax.dot_general(q, wsox_ref[...], dn,
                        preferred_element_type=jnp.float32) + bsox_ref[...]
    Y = lax.dot_general(q, wsoy_ref[...], dn,
                        preferred_element_type=jnp.float32) + bsoy_ref[...]
    logits = lax.dot_general(q, waw_ref[...], dn,
                             preferred_element_type=jnp.float32) + baw_ref[...]
    l0 = logits[:, 0:8]
    l1 = logits[:, 8:16]
    e0 = jnp.exp(l0 - jnp.max(l0, axis=1, keepdims=True))
    e1 = jnp.exp(l1 - jnp.max(l1, axis=1, keepdims=True))
    a0 = e0 / jnp.sum(e0, axis=1, keepdims=True)
    a1 = e1 / jnp.sum(e1, axis=1, keepdims=True)
    aw = jnp.concatenate([a0, a1], axis=1)

    wl = jnp.asarray(_W_L).reshape(1, _NCOMBO)
    hl = jnp.asarray(_H_L).reshape(1, _NCOMBO)
    base = (jnp.asarray(_START_L).reshape(1, _NCOMBO)
            + (b * _NV).astype(jnp.int32))
    wdim = wl.astype(jnp.int32)

    def expand(col, scale64, scale32):
        c64 = jnp.broadcast_to(ql_ref[:, col:col + 1] * scale64, (_BQ, 4))
        c32 = jnp.broadcast_to(ql_ref[:, col + 2:col + 3] * scale32, (_BQ, 4))
        return jnp.concatenate([c64, c32, c64, c32], axis=1)

    x = X + expand(0, float(_SPATIAL[0][1]), float(_SPATIAL[1][1])) - 0.5
    y = Y + expand(1, float(_SPATIAL[0][0]), float(_SPATIAL[1][0])) - 0.5

    x0 = jnp.floor(x)
    y0 = jnp.floor(y)
    fx1 = x - x0
    fx0 = 1.0 - fx1
    fy1 = y - y0
    fy0 = 1.0 - fy1

    idxs = []
    wts = []
    for dy, fy in ((0.0, fy0), (1.0, fy1)):
        yi = y0 + dy
        for dx, fx in ((0.0, fx0), (1.0, fx1)):
            xi = x0 + dx
            valid = ((xi >= 0.0) & (xi <= wl - 1.0)
                     & (yi >= 0.0) & (yi <= hl - 1.0))
            xc = jnp.clip(xi, 0.0, wl - 1.0).astype(jnp.int32)
            yc = jnp.clip(yi, 0.0, hl - 1.0).astype(jnp.int32)
            idxs.append(base + yc * wdim + xc)
            wts.append(aw * fx * fy * valid.astype(jnp.float32))
    idx_ref[...] = jnp.stack(idxs, axis=-1).reshape(_BQ, _NCOMBO * 4)
    wt_ref[...] = jnp.stack(wts, axis=-1).reshape(_BQ, _NCOMBO * 4)


def _proj_body(a_ref, w_ref, b_ref, o_ref):
    o_ref[...] = lax.dot_general(
        a_ref[...], w_ref[...], (((1,), (1,)), ((), ())),
        preferred_element_type=jnp.float32) + b_ref[...]


_NW = 32
_PER_W = _NOUT // _NW
_CH = 8
_ROWS_CH = _CH * _K
_NCHUNK = _PER_W // _CH
_IDXW = 128


def _sc_attend(value_flat, idx2d, wts_flat):
    mesh = plsc.VectorSubcoreMesh(core_axis_name="c", subcore_axis_name="s")
    idx_rows_w = _PER_W * _K // _IDXW

    @functools.partial(
        pl.kernel,
        out_type=jax.ShapeDtypeStruct((_NOUT, _D), jnp.float32),
        mesh=mesh,
        scratch_types=[
            pltpu.VMEM((idx_rows_w, _IDXW), jnp.int32),
            pltpu.VMEM((_PER_W * _K,), jnp.float32),
            pltpu.VMEM((_ROWS_CH, _D), jnp.float32),
            pltpu.VMEM((_CH, _D), jnp.float32),
            pltpu.SemaphoreType.DMA,
        ],
    )
    def k(value_hbm, idx_hbm, wts_hbm, out_hbm, idx_v, wts_v, rows_v, out_v,
          sem):
        wid = lax.axis_index("s") * 2 + lax.axis_index("c")
        out_base = wid * _PER_W
        pltpu.sync_copy(idx_hbm.at[pl.ds(wid * idx_rows_w, idx_rows_w)], idx_v)
        pltpu.sync_copy(wts_hbm.at[pl.ds(out_base * _K, _PER_W * _K)], wts_v)

        gathers_per_chunk = _ROWS_CH // _IDXW

        def chunk(g, carry):
            copies = []
            for j in range(gathers_per_chunk):
                copies.append(pltpu.async_copy(
                    value_hbm.at[idx_v.at[g * gathers_per_chunk + j]],
                    rows_v.at[pl.ds(j * _IDXW, _IDXW)], sem))
            for cp in copies:
                cp.wait()
            wbase = g * _ROWS_CH
            for o in range(_CH):
                def rbody(r, acc):
                    wvec = plsc.load_gather(
                        wts_v,
                        [jnp.full((16,), wbase + o * _K + r, jnp.int32)])
                    row = o * _K + r
                    return tuple(
                        acc[c] + wvec * rows_v[row, pl.ds(c * 16, 16)]
                        for c in range(16))
                acc = lax.fori_loop(
                    0, _K, rbody,
                    tuple(jnp.zeros((16,), jnp.float32) for _ in range(16)))
                for c in range(16):
                    out_v[o, pl.ds(c * 16, 16)] = acc[c]
            pltpu.sync_copy(out_v, out_hbm.at[pl.ds(out_base + g * _CH, _CH)])
            return carry

        lax.fori_loop(0, _NCHUNK, chunk, 0)

    return k(value_flat, idx2d, wts_flat)


def _prep_call(query, query_location, W_so, b_so, W_aw, b_aw):
    q2 = query.reshape(_BS * _NQ, _D)
    ql2 = query_location.reshape(_BS * _NQ, _L * 2)
    grid = (_BS * _NQ // _BQ,)
    full = lambda i: (0, 0)
    row = lambda i: (i, 0)
    return pl.pallas_call(
        _prep_body,
        grid=grid,
        in_specs=[
            pl.BlockSpec((_BQ, _D), row),
            pl.BlockSpec((_BQ, _L * 2), row),
            pl.BlockSpec((_NCOMBO, _D), full),
            pl.BlockSpec((_NCOMBO, _D), full),
            pl.BlockSpec((1, _NCOMBO), full),
            pl.BlockSpec((1, _NCOMBO), full),
            pl.BlockSpec((_NCOMBO, _D), full),
            pl.BlockSpec((1, _NCOMBO), full),
        ],
        out_specs=[
            pl.BlockSpec((_BQ, _NCOMBO * 4), row),
            pl.BlockSpec((_BQ, _NCOMBO * 4), row),
        ],
        out_shape=[
            jax.ShapeDtypeStruct((_BS * _NQ, _NCOMBO * 4), jnp.int32),
            jax.ShapeDtypeStruct((_BS * _NQ, _NCOMBO * 4), jnp.float32),
        ],
    )(q2, ql2,
      W_so[0::2], W_so[1::2],
      b_so[0::2].reshape(1, _NCOMBO), b_so[1::2].reshape(1, _NCOMBO),
      W_aw, b_aw.reshape(1, _NCOMBO))


def _proj_call(attn2, W_op, b_op):
    grid = (_BS * _NQ // _BQ,)
    return pl.pallas_call(
        _proj_body,
        grid=grid,
        in_specs=[
            pl.BlockSpec((_BQ, _H * _D), lambda i: (i, 0)),
            pl.BlockSpec((_D, _H * _D), lambda i: (0, 0)),
            pl.BlockSpec((1, _D), lambda i: (0, 0)),
        ],
        out_specs=pl.BlockSpec((_BQ, _D), lambda i: (i, 0)),
        out_shape=jax.ShapeDtypeStruct((_BS * _NQ, _D), jnp.float32),
    )(attn2, W_op, b_op.reshape(1, _D))


def kernel(query, value, query_location, spatial_shapes, level_start_index,
           W_so, b_so, W_aw, b_aw, W_op, b_op):
    idx, wt = _prep_call(query, query_location, W_so, b_so, W_aw, b_aw)
    value_flat = value.reshape(_BS * _NV, _D)
    idx2d = idx.reshape(-1, _IDXW)
    attn = _sc_attend(value_flat, idx2d, wt.reshape(-1))
    attn2 = attn.reshape(_BS * _NQ, _H * _D)
    out = _proj_call(attn2, W_op, b_op)
    return out.reshape(_BS, _NQ, _D)


# speedup vs baseline: 14.5203x; 14.5203x over previous
import functools
import numpy as np
import jax
import jax.numpy as jnp
from jax import lax
from jax.experimental import pallas as pl
from jax.experimental.pallas import tpu as pltpu
from jax.experimental.pallas import tpu_sc as plsc

_D = 256
_H = 2
_L = 2
_P = 4
_SPATIAL = ((64, 64), (32, 32))
_BS = 2
_NQ = _SPATIAL[0][0] * _SPATIAL[0][1]
_NV = sum(h * w for h, w in _SPATIAL)
_NCOMBO = _H * _L * _P
_K = _L * _P * 4
_NOUT = _BS * _NQ * _H

_BQ = 512

_combo_l = np.array([(c % 8) // 4 for c in range(_NCOMBO)])
_W_L = np.array([_SPATIAL[l][1] for l in _combo_l], np.float32)
_H_L = np.array([_SPATIAL[l][0] for l in _combo_l], np.float32)
_START_L = np.array([0 if l == 0 else _SPATIAL[0][0] * _SPATIAL[0][1]
                     for l in _combo_l], np.int32)


def _prep_body(q_ref, ql_ref, wsox_ref, wsoy_ref, bsox_ref, bsoy_ref,
               waw_ref, baw_ref, idx_ref, wt_ref):
    b = pl.program_id(0) // (_NQ // _BQ)
    q = q_ref[...]
    dn = (((1,), (1,)), ((), ()))
    X = lax.dot_general(q, wsox_ref[...], dn,
                        preferred_element_type=jnp.float32) + bsox_ref[...]
    Y = lax.dot_general(q, wsoy_ref[...], dn,
                        preferred_element_type=jnp.float32) + bsoy_ref[...]
    logits = lax.dot_general(q, waw_ref[...], dn,
                             preferred_element_type=jnp.float32) + baw_ref[...]
    l0 = logits[:, 0:8]
    l1 = logits[:, 8:16]
    e0 = jnp.exp(l0 - jnp.max(l0, axis=1, keepdims=True))
    e1 = jnp.exp(l1 - jnp.max(l1, axis=1, keepdims=True))
    a0 = e0 / jnp.sum(e0, axis=1, keepdims=True)
    a1 = e1 / jnp.sum(e1, axis=1, keepdims=True)
    aw = jnp.concatenate([a0, a1], axis=1)

    def combo_const(v0, v1, dtype):
        return jnp.concatenate(
            [jnp.full((1, 4), v0, dtype), jnp.full((1, 4), v1, dtype)] * 2,
            axis=1)

    wl = combo_const(_SPATIAL[0][1], _SPATIAL[1][1], jnp.float32)
    hl = combo_const(_SPATIAL[0][0], _SPATIAL[1][0], jnp.float32)
    base = (combo_const(0, _SPATIAL[0][0] * _SPATIAL[0][1], jnp.int32)
            + (b * _NV).astype(jnp.int32))
    wdim = wl.astype(jnp.int32)

    def expand(col, scale64, scale32):
        c64 = jnp.broadcast_to(ql_ref[:, col:col + 1] * scale64, (_BQ, 4))
        c32 = jnp.broadcast_to(ql_ref[:, col + 2:col + 3] * scale32, (_BQ, 4))
        return jnp.concatenate([c64, c32, c64, c32], axis=1)

    x = X + expand(0, float(_SPATIAL[0][1]), float(_SPATIAL[1][1])) - 0.5
    y = Y + expand(1, float(_SPATIAL[0][0]), float(_SPATIAL[1][0])) - 0.5

    x0 = jnp.floor(x)
    y0 = jnp.floor(y)
    fx1 = x - x0
    fx0 = 1.0 - fx1
    fy1 = y - y0
    fy0 = 1.0 - fy1

    idxs = []
    wts = []
    for dy, fy in ((0.0, fy0), (1.0, fy1)):
        yi = y0 + dy
        for dx, fx in ((0.0, fx0), (1.0, fx1)):
            xi = x0 + dx
            valid = ((xi >= 0.0) & (xi <= wl - 1.0)
                     & (yi >= 0.0) & (yi <= hl - 1.0))
            xc = jnp.clip(xi, 0.0, wl - 1.0).astype(jnp.int32)
            yc = jnp.clip(yi, 0.0, hl - 1.0).astype(jnp.int32)
            idxs.append(base + yc * wdim + xc)
            wts.append(aw * fx * fy * valid.astype(jnp.float32))
    idx_ref[...] = jnp.stack(idxs, axis=-1).reshape(_BQ, _NCOMBO * 4)
    wt_ref[...] = jnp.stack(wts, axis=-1).reshape(_BQ, _NCOMBO * 4)


def _proj_body(a_ref, w_ref, b_ref, o_ref):
    o_ref[...] = lax.dot_general(
        a_ref[...], w_ref[...], (((1,), (1,)), ((), ())),
        preferred_element_type=jnp.float32) + b_ref[...]


_NW = 32
_PER_W = _NOUT // _NW
_CH = 8
_ROWS_CH = _CH * _K
_NCHUNK = _PER_W // _CH
_IDXW = 128


def _sc_attend(value_flat, idx2d, wts_flat):
    mesh = plsc.VectorSubcoreMesh(core_axis_name="c", subcore_axis_name="s")
    idx_rows_w = _PER_W * _K // _IDXW

    @functools.partial(
        pl.kernel,
        out_type=jax.ShapeDtypeStruct((_NOUT, _D), jnp.float32),
        mesh=mesh,
        scratch_types=[
            pltpu.VMEM((idx_rows_w, _IDXW), jnp.int32),
            pltpu.VMEM((_PER_W * _K,), jnp.float32),
            pltpu.VMEM((_ROWS_CH, _D), jnp.float32),
            pltpu.VMEM((_CH, _D), jnp.float32),
            pltpu.SemaphoreType.DMA,
        ],
        compiler_params=pltpu.CompilerParams(needs_layout_passes=False),
    )
    def k(value_hbm, idx_hbm, wts_hbm, out_hbm, idx_v, wts_v, rows_v, out_v,
          sem):
        wid = lax.axis_index("s") * 2 + lax.axis_index("c")
        out_base = wid * _PER_W
        pltpu.sync_copy(idx_hbm.at[pl.ds(wid * idx_rows_w, idx_rows_w)], idx_v)
        pltpu.sync_copy(wts_hbm.at[pl.ds(out_base * _K, _PER_W * _K)], wts_v)

        gathers_per_chunk = _ROWS_CH // _IDXW

        def chunk(g, carry):
            copies = []
            for j in range(gathers_per_chunk):
                copies.append(pltpu.async_copy(
                    value_hbm.at[idx_v.at[g * gathers_per_chunk + j]],
                    rows_v.at[pl.ds(j * _IDXW, _IDXW)], sem))
            for cp in copies:
                cp.wait()
            wbase = g * _ROWS_CH
            for o in range(_CH):
                def rbody(r, acc):
                    wvec = plsc.load_gather(
                        wts_v,
                        [jnp.full((16,), wbase + o * _K + r, jnp.int32)])
                    row = o * _K + r
                    return tuple(
                        acc[c] + wvec * rows_v[row, pl.ds(c * 16, 16)]
                        for c in range(16))
                acc = lax.fori_loop(
                    0, _K, rbody,
                    tuple(jnp.zeros((16,), jnp.float32) for _ in range(16)))
                for c in range(16):
                    out_v[o, pl.ds(c * 16, 16)] = acc[c]
            pltpu.sync_copy(out_v, out_hbm.at[pl.ds(out_base + g * _CH, _CH)])
            return carry

        lax.fori_loop(0, _NCHUNK, chunk, 0)

    return k(value_flat, idx2d, wts_flat)


def _prep_call(query, query_location, W_so, b_so, W_aw, b_aw):
    q2 = query.reshape(_BS * _NQ, _D)
    ql2 = query_location.reshape(_BS * _NQ, _L * 2)
    grid = (_BS * _NQ // _BQ,)
    full = lambda i: (0, 0)
    row = lambda i: (i, 0)
    return pl.pallas_call(
        _prep_body,
        grid=grid,
        in_specs=[
            pl.BlockSpec((_BQ, _D), row),
            pl.BlockSpec((_BQ, _L * 2), row),
            pl.BlockSpec((_NCOMBO, _D), full),
            pl.BlockSpec((_NCOMBO, _D), full),
            pl.BlockSpec((1, _NCOMBO), full),
            pl.BlockSpec((1, _NCOMBO), full),
            pl.BlockSpec((_NCOMBO, _D), full),
            pl.BlockSpec((1, _NCOMBO), full),
        ],
        out_specs=[
            pl.BlockSpec((_BQ, _NCOMBO * 4), row),
            pl.BlockSpec((_BQ, _NCOMBO * 4), row),
        ],
        out_shape=[
            jax.ShapeDtypeStruct((_BS * _NQ, _NCOMBO * 4), jnp.int32),
            jax.ShapeDtypeStruct((_BS * _NQ, _NCOMBO * 4), jnp.float32),
        ],
    )(q2, ql2,
      W_so[0::2], W_so[1::2],
      b_so[0::2].reshape(1, _NCOMBO), b_so[1::2].reshape(1, _NCOMBO),
      W_aw, b_aw.reshape(1, _NCOMBO))


def _proj_call(attn2, W_op, b_op):
    grid = (_BS * _NQ // _BQ,)
    return pl.pallas_call(
        _proj_body,
        grid=grid,
        in_specs=[
            pl.BlockSpec((_BQ, _H * _D), lambda i: (i, 0)),
            pl.BlockSpec((_D, _H * _D), lambda i: (0, 0)),
            pl.BlockSpec((1, _D), lambda i: (0, 0)),
        ],
        out_specs=pl.BlockSpec((_BQ, _D), lambda i: (i, 0)),
        out_shape=jax.ShapeDtypeStruct((_BS * _NQ, _D), jnp.float32),
    )(attn2, W_op, b_op.reshape(1, _D))


def kernel(query, value, query_location, spatial_shapes, level_start_index,
           W_so, b_so, W_aw, b_aw, W_op, b_op):
    idx, wt = _prep_call(query, query_location, W_so, b_so, W_aw, b_aw)
    value_flat = value.reshape(_BS * _NV, _D)
    idx2d = idx.reshape(-1, _IDXW)
    attn = _sc_attend(value_flat, idx2d, wt.reshape(-1))
    attn2 = attn.reshape(_BS * _NQ, _H * _D)
    out = _proj_call(attn2, W_op, b_op)
    return out.reshape(_BS, _NQ, _D)


# speedup vs baseline: 25.7622x; 1.7742x over previous
import functools
import numpy as np
import jax
import jax.numpy as jnp
from jax import lax
from jax.experimental import pallas as pl
from jax.experimental.pallas import tpu as pltpu
from jax.experimental.pallas import tpu_sc as plsc

_D = 256
_H = 2
_L = 2
_P = 4
_SPATIAL = ((64, 64), (32, 32))
_BS = 2
_NQ = _SPATIAL[0][0] * _SPATIAL[0][1]
_NV = sum(h * w for h, w in _SPATIAL)
_NCOMBO = _H * _L * _P
_K = _L * _P * 4
_NOUT = _BS * _NQ * _H

_BQ = 512

_combo_l = np.array([(c % 8) // 4 for c in range(_NCOMBO)])
_W_L = np.array([_SPATIAL[l][1] for l in _combo_l], np.float32)
_H_L = np.array([_SPATIAL[l][0] for l in _combo_l], np.float32)
_START_L = np.array([0 if l == 0 else _SPATIAL[0][0] * _SPATIAL[0][1]
                     for l in _combo_l], np.int32)


def _prep_body(q_ref, ql_ref, wsox_ref, wsoy_ref, bsox_ref, bsoy_ref,
               waw_ref, baw_ref, *out_refs):
    idx_refs = out_refs[:4]
    wt_refs = out_refs[4:]
    b = pl.program_id(0) // (_NQ // _BQ)
    q = q_ref[...]
    dn = (((1,), (1,)), ((), ()))
    X = lax.dot_general(q, wsox_ref[...], dn,
                        preferred_element_type=jnp.float32) + bsox_ref[...]
    Y = lax.dot_general(q, wsoy_ref[...], dn,
                        preferred_element_type=jnp.float32) + bsoy_ref[...]
    logits = lax.dot_general(q, waw_ref[...], dn,
                             preferred_element_type=jnp.float32) + baw_ref[...]
    l0 = logits[:, 0:8]
    l1 = logits[:, 8:16]
    e0 = jnp.exp(l0 - jnp.max(l0, axis=1, keepdims=True))
    e1 = jnp.exp(l1 - jnp.max(l1, axis=1, keepdims=True))
    a0 = e0 / jnp.sum(e0, axis=1, keepdims=True)
    a1 = e1 / jnp.sum(e1, axis=1, keepdims=True)
    aw = jnp.concatenate([a0, a1], axis=1)

    def combo_const(v0, v1, dtype):
        return jnp.concatenate(
            [jnp.full((1, 4), v0, dtype), jnp.full((1, 4), v1, dtype)] * 2,
            axis=1)

    wl = combo_const(_SPATIAL[0][1], _SPATIAL[1][1], jnp.float32)
    hl = combo_const(_SPATIAL[0][0], _SPATIAL[1][0], jnp.float32)
    base = (combo_const(0, _SPATIAL[0][0] * _SPATIAL[0][1], jnp.int32)
            + (b * _NV).astype(jnp.int32))
    wdim = wl.astype(jnp.int32)

    def expand(col, scale64, scale32):
        c64 = jnp.broadcast_to(ql_ref[:, col:col + 1] * scale64, (_BQ, 4))
        c32 = jnp.broadcast_to(ql_ref[:, col + 2:col + 3] * scale32, (_BQ, 4))
        return jnp.concatenate([c64, c32, c64, c32], axis=1)

    x = X + expand(0, float(_SPATIAL[0][1]), float(_SPATIAL[1][1])) - 0.5
    y = Y + expand(1, float(_SPATIAL[0][0]), float(_SPATIAL[1][0])) - 0.5

    x0 = jnp.floor(x)
    y0 = jnp.floor(y)
    fx1 = x - x0
    fx0 = 1.0 - fx1
    fy1 = y - y0
    fy0 = 1.0 - fy1

    j = 0
    for dy, fy in ((0.0, fy0), (1.0, fy1)):
        yi = y0 + dy
        for dx, fx in ((0.0, fx0), (1.0, fx1)):
            xi = x0 + dx
            valid = ((xi >= 0.0) & (xi <= wl - 1.0)
                     & (yi >= 0.0) & (yi <= hl - 1.0))
            xc = jnp.clip(xi, 0.0, wl - 1.0).astype(jnp.int32)
            yc = jnp.clip(yi, 0.0, hl - 1.0).astype(jnp.int32)
            idx_refs[j][...] = base + yc * wdim + xc
            wt_refs[j][...] = aw * fx * fy * valid.astype(jnp.float32)
            j += 1


def _proj_body(a_ref, w_ref, b_ref, o_ref):
    o_ref[...] = lax.dot_general(
        a_ref[...], w_ref[...], (((1,), (1,)), ((), ())),
        preferred_element_type=jnp.float32) + b_ref[...]


_NW = 32
_PER_W = _NOUT // _NW
_KC = _L * _P
_CH = 4
_IDXC = _CH * _KC
_ROWS_CH = 4 * _IDXC
_NCHUNK = _PER_W // _CH


def _sc_attend(value_flat, idxs, wtss):
    mesh = plsc.VectorSubcoreMesh(core_axis_name="c", subcore_axis_name="s")

    @functools.partial(
        pl.kernel,
        out_type=jax.ShapeDtypeStruct((_NOUT, _D), jnp.float32),
        mesh=mesh,
        scratch_types=(
            [pltpu.VMEM((_NCHUNK * _IDXC,), jnp.int32)] * 4
            + [pltpu.VMEM((_PER_W * _KC,), jnp.float32)] * 4
            + [pltpu.VMEM((_ROWS_CH, _D), jnp.float32)] * 2
            + [pltpu.VMEM((_CH, _D), jnp.float32),
               pltpu.SemaphoreType.DMA, pltpu.SemaphoreType.DMA]
        ),
        compiler_params=pltpu.CompilerParams(needs_layout_passes=False),
    )
    def k(value_hbm, i0, i1, i2, i3, w0, w1, w2, w3, out_hbm,
          iv0, iv1, iv2, iv3, wv0, wv1, wv2, wv3, bufa, bufb, out_v,
          sema, semb):
        idx_hbm = (i0, i1, i2, i3)
        wts_hbm = (w0, w1, w2, w3)
        idx_v = (iv0, iv1, iv2, iv3)
        wts_v = (wv0, wv1, wv2, wv3)
        wid = lax.axis_index("s") * 2 + lax.axis_index("c")
        out_base = wid * _PER_W
        for j in range(4):
            pltpu.sync_copy(idx_hbm[j].at[pl.ds(wid * _PER_W * _KC,
                                                _PER_W * _KC)], idx_v[j])
            pltpu.sync_copy(wts_hbm[j].at[pl.ds(wid * _PER_W * _KC,
                                                _PER_W * _KC)], wts_v[j])

        def start(g, buf, sem):
            for j in range(4):
                pltpu.async_copy(
                    value_hbm.at[idx_v[j].at[pl.ds(g * _IDXC, _IDXC)]],
                    buf.at[pl.ds(j * _IDXC, _IDXC)], sem)

        def drain(buf, sem):
            pltpu.make_async_copy(value_hbm.at[pl.ds(0, _ROWS_CH)], buf,
                                  sem).wait()

        def compute(g, buf):
            for o in range(_CH):
                def mbody(m, acc):
                    for j in range(4):
                        wvec = plsc.load_gather(
                            wts_v[j],
                            [jnp.full((16,), g * _IDXC + o * _KC + m,
                                      jnp.int32)])
                        row = j * _IDXC + o * _KC + m
                        acc = tuple(
                            acc[c] + wvec * buf[row, pl.ds(c * 16, 16)]
                            for c in range(16))
                    return acc
                acc = lax.fori_loop(
                    0, _KC, mbody,
                    tuple(jnp.zeros((16,), jnp.float32) for _ in range(16)))
                for c in range(16):
                    out_v[o, pl.ds(c * 16, 16)] = acc[c]
            pltpu.sync_copy(out_v, out_hbm.at[pl.ds(out_base + g * _CH, _CH)])

        start(0, bufa, sema)

        def pair(gp, carry):
            g0 = gp * 2
            start(g0 + 1, bufb, semb)
            drain(bufa, sema)
            compute(g0, bufa)

            @pl.when(g0 + 2 < _NCHUNK)
            def _():
                start(g0 + 2, bufa, sema)

            drain(bufb, semb)
            compute(g0 + 1, bufb)
            return carry

        lax.fori_loop(0, _NCHUNK // 2, pair, 0)

    return k(value_flat, *idxs, *wtss)


def _prep_call(query, query_location, W_so, b_so, W_aw, b_aw):
    q2 = query.reshape(_BS * _NQ, _D)
    ql2 = query_location.reshape(_BS * _NQ, _L * 2)
    grid = (_BS * _NQ // _BQ,)
    full = lambda i: (0, 0)
    row = lambda i: (i, 0)
    return pl.pallas_call(
        _prep_body,
        grid=grid,
        in_specs=[
            pl.BlockSpec((_BQ, _D), row),
            pl.BlockSpec((_BQ, _L * 2), row),
            pl.BlockSpec((_NCOMBO, _D), full),
            pl.BlockSpec((_NCOMBO, _D), full),
            pl.BlockSpec((1, _NCOMBO), full),
            pl.BlockSpec((1, _NCOMBO), full),
            pl.BlockSpec((_NCOMBO, _D), full),
            pl.BlockSpec((1, _NCOMBO), full),
        ],
        out_specs=[pl.BlockSpec((_BQ, _NCOMBO), row)] * 8,
        out_shape=(
            [jax.ShapeDtypeStruct((_BS * _NQ, _NCOMBO), jnp.int32)] * 4
            + [jax.ShapeDtypeStruct((_BS * _NQ, _NCOMBO), jnp.float32)] * 4
        ),
    )(q2, ql2,
      W_so[0::2], W_so[1::2],
      b_so[0::2].reshape(1, _NCOMBO), b_so[1::2].reshape(1, _NCOMBO),
      W_aw, b_aw.reshape(1, _NCOMBO))


def _proj_call(attn2, W_op, b_op):
    grid = (_BS * _NQ // _BQ,)
    return pl.pallas_call(
        _proj_body,
        grid=grid,
        in_specs=[
            pl.BlockSpec((_BQ, _H * _D), lambda i: (i, 0)),
            pl.BlockSpec((_D, _H * _D), lambda i: (0, 0)),
            pl.BlockSpec((1, _D), lambda i: (0, 0)),
        ],
        out_specs=pl.BlockSpec((_BQ, _D), lambda i: (i, 0)),
        out_shape=jax.ShapeDtypeStruct((_BS * _NQ, _D), jnp.float32),
    )(attn2, W_op, b_op.reshape(1, _D))


def kernel(query, value, query_location, spatial_shapes, level_start_index,
           W_so, b_so, W_aw, b_aw, W_op, b_op):
    outs = _prep_call(query, query_location, W_so, b_so, W_aw, b_aw)
    value_flat = value.reshape(_BS * _NV, _D)
    idxs = [o.reshape(-1) for o in outs[:4]]
    wtss = [o.reshape(-1) for o in outs[4:]]
    attn = _sc_attend(value_flat, idxs, wtss)
    attn2 = attn.reshape(_BS * _NQ, _H * _D)
    out = _proj_call(attn2, W_op, b_op)
    return out.reshape(_BS, _NQ, _D)


# speedup vs baseline: 28.2775x; 1.0976x over previous
import functools
import numpy as np
import jax
import jax.numpy as jnp
from jax import lax
from jax.experimental import pallas as pl
from jax.experimental.pallas import tpu as pltpu
from jax.experimental.pallas import tpu_sc as plsc

_D = 256
_H = 2
_L = 2
_P = 4
_SPATIAL = ((64, 64), (32, 32))
_BS = 2
_NQ = _SPATIAL[0][0] * _SPATIAL[0][1]
_NV = sum(h * w for h, w in _SPATIAL)
_NCOMBO = _H * _L * _P
_K = _L * _P * 4
_NOUT = _BS * _NQ * _H

_BQ = 512

_combo_l = np.array([(c % 8) // 4 for c in range(_NCOMBO)])
_W_L = np.array([_SPATIAL[l][1] for l in _combo_l], np.float32)
_H_L = np.array([_SPATIAL[l][0] for l in _combo_l], np.float32)
_START_L = np.array([0 if l == 0 else _SPATIAL[0][0] * _SPATIAL[0][1]
                     for l in _combo_l], np.int32)


def _prep_body(q_ref, ql_ref, wsox_ref, wsoy_ref, bsox_ref, bsoy_ref,
               waw_ref, baw_ref, idx_ref, wt_ref):
    b = pl.program_id(0) // (_NQ // _BQ)
    q = q_ref[...]
    dn = (((1,), (1,)), ((), ()))
    X = lax.dot_general(q, wsox_ref[...], dn,
                        preferred_element_type=jnp.float32) + bsox_ref[...]
    Y = lax.dot_general(q, wsoy_ref[...], dn,
                        preferred_element_type=jnp.float32) + bsoy_ref[...]
    logits = lax.dot_general(q, waw_ref[...], dn,
                             preferred_element_type=jnp.float32) + baw_ref[...]
    l0 = logits[:, 0:8]
    l1 = logits[:, 8:16]
    e0 = jnp.exp(l0 - jnp.max(l0, axis=1, keepdims=True))
    e1 = jnp.exp(l1 - jnp.max(l1, axis=1, keepdims=True))
    a0 = e0 / jnp.sum(e0, axis=1, keepdims=True)
    a1 = e1 / jnp.sum(e1, axis=1, keepdims=True)
    aw = jnp.concatenate([a0, a1], axis=1)

    def combo_const(v0, v1, dtype):
        return jnp.concatenate(
            [jnp.full((1, 4), v0, dtype), jnp.full((1, 4), v1, dtype)] * 2,
            axis=1)

    wl = combo_const(_SPATIAL[0][1], _SPATIAL[1][1], jnp.float32)
    hl = combo_const(_SPATIAL[0][0], _SPATIAL[1][0], jnp.float32)
    base = (combo_const(0, _SPATIAL[0][0] * _SPATIAL[0][1], jnp.int32)
            + (b * _NV).astype(jnp.int32))
    wdim = wl.astype(jnp.int32)

    def expand(col, scale64, scale32):
        c64 = jnp.broadcast_to(ql_ref[:, col:col + 1] * scale64, (_BQ, 4))
        c32 = jnp.broadcast_to(ql_ref[:, col + 2:col + 3] * scale32, (_BQ, 4))
        return jnp.concatenate([c64, c32, c64, c32], axis=1)

    x = X + expand(0, float(_SPATIAL[0][1]), float(_SPATIAL[1][1])) - 0.5
    y = Y + expand(1, float(_SPATIAL[0][0]), float(_SPATIAL[1][0])) - 0.5

    x0 = jnp.floor(x)
    y0 = jnp.floor(y)
    fx1 = x - x0
    fx0 = 1.0 - fx1
    fy1 = y - y0
    fy0 = 1.0 - fy1

    idxs = []
    wts = []
    for dy, fy in ((0.0, fy0), (1.0, fy1)):
        yi = y0 + dy
        for dx, fx in ((0.0, fx0), (1.0, fx1)):
            xi = x0 + dx
            valid = ((xi >= 0.0) & (xi <= wl - 1.0)
                     & (yi >= 0.0) & (yi <= hl - 1.0))
            xc = jnp.clip(xi, 0.0, wl - 1.0).astype(jnp.int32)
            yc = jnp.clip(yi, 0.0, hl - 1.0).astype(jnp.int32)
            idxs.append(base + yc * wdim + xc)
            wts.append(aw * fx * fy * valid.astype(jnp.float32))
    idx_ref[...] = jnp.concatenate(idxs, axis=1)
    wt_ref[...] = jnp.concatenate(wts, axis=1)


def _proj_body(a_ref, w_ref, b_ref, o_ref):
    o_ref[...] = lax.dot_general(
        a_ref[...], w_ref[...], (((1,), (1,)), ((), ())),
        preferred_element_type=jnp.float32) + b_ref[...]


_NW = 32
_KC = _L * _P
_QROWS = _BS * _NQ
_PER_W2 = _QROWS // _NW
_CHQ = 2
_ROWS_CH = _CHQ * _H * 4 * _KC
_NCHUNK = _PER_W2 // _CHQ
_ENT_W = _PER_W2 * _H * 4 * _KC


def _sc_attend(value_flat, idx_flat, wts_flat):
    mesh = plsc.VectorSubcoreMesh(core_axis_name="c", subcore_axis_name="s")

    @functools.partial(
        pl.kernel,
        out_type=jax.ShapeDtypeStruct((_QROWS, _H * _D), jnp.float32),
        mesh=mesh,
        scratch_types=[
            pltpu.VMEM((_ENT_W,), jnp.int32),
            pltpu.VMEM((_ENT_W,), jnp.float32),
            pltpu.VMEM((_ROWS_CH, _D), jnp.float32),
            pltpu.VMEM((_ROWS_CH, _D), jnp.float32),
            pltpu.VMEM((_CHQ, _H * _D), jnp.float32),
            pltpu.SemaphoreType.DMA,
            pltpu.SemaphoreType.DMA,
        ],
        compiler_params=pltpu.CompilerParams(needs_layout_passes=False),
    )
    def k(value_hbm, idx_hbm, wts_hbm, out_hbm, idx_v, wts_v, bufa, bufb,
          out_v, sema, semb):
        wid = lax.axis_index("s") * 2 + lax.axis_index("c")
        out_base = wid * _PER_W2
        pltpu.sync_copy(idx_hbm.at[pl.ds(wid * _ENT_W, _ENT_W)], idx_v)
        pltpu.sync_copy(wts_hbm.at[pl.ds(wid * _ENT_W, _ENT_W)], wts_v)

        def start(g, buf, sem):
            pltpu.async_copy(
                value_hbm.at[idx_v.at[pl.ds(g * _ROWS_CH, _ROWS_CH)]],
                buf, sem)

        def drain(buf, sem):
            pltpu.make_async_copy(value_hbm.at[pl.ds(0, _ROWS_CH)], buf,
                                  sem).wait()

        def compute(g, buf):
            for q2 in range(_CHQ):
                for h in range(_H):
                    def mbody(jm, acc, q2=q2, h=h):
                        j = jm // _KC
                        m = jm % _KC
                        row = q2 * 64 + j * 16 + h * _KC + m
                        wvec = plsc.load_gather(
                            wts_v,
                            [jnp.full((16,), g * _ROWS_CH + row, jnp.int32)])
                        return tuple(
                            acc[c] + wvec * buf[row, pl.ds(c * 16, 16)]
                            for c in range(16))
                    acc = lax.fori_loop(
                        0, 4 * _KC, mbody,
                        tuple(jnp.zeros((16,), jnp.float32)
                              for _ in range(16)))
                    for c in range(16):
                        out_v[q2, pl.ds(h * _D + c * 16, 16)] = acc[c]
            pltpu.sync_copy(out_v,
                            out_hbm.at[pl.ds(out_base + g * _CHQ, _CHQ)])

        start(0, bufa, sema)

        def pair(gp, carry):
            g0 = gp * 2
            start(g0 + 1, bufb, semb)
            drain(bufa, sema)
            compute(g0, bufa)

            @pl.when(g0 + 2 < _NCHUNK)
            def _():
                start(g0 + 2, bufa, sema)

            drain(bufb, semb)
            compute(g0 + 1, bufb)
            return carry

        lax.fori_loop(0, _NCHUNK // 2, pair, 0)

    return k(value_flat, idx_flat, wts_flat)


def _prep_call(query, query_location, W_so, b_so, W_aw, b_aw):
    q2 = query.reshape(_BS * _NQ, _D)
    ql2 = query_location.reshape(_BS * _NQ, _L * 2)
    grid = (_BS * _NQ // _BQ,)
    full = lambda i: (0, 0)
    row = lambda i: (i, 0)
    return pl.pallas_call(
        _prep_body,
        grid=grid,
        in_specs=[
            pl.BlockSpec((_BQ, _D), row),
            pl.BlockSpec((_BQ, _L * 2), row),
            pl.BlockSpec((_NCOMBO, _D), full),
            pl.BlockSpec((_NCOMBO, _D), full),
            pl.BlockSpec((1, _NCOMBO), full),
            pl.BlockSpec((1, _NCOMBO), full),
            pl.BlockSpec((_NCOMBO, _D), full),
            pl.BlockSpec((1, _NCOMBO), full),
        ],
        out_specs=[pl.BlockSpec((_BQ, _NCOMBO * 4), row)] * 2,
        out_shape=[
            jax.ShapeDtypeStruct((_BS * _NQ, _NCOMBO * 4), jnp.int32),
            jax.ShapeDtypeStruct((_BS * _NQ, _NCOMBO * 4), jnp.float32),
        ],
    )(q2, ql2,
      W_so[0::2], W_so[1::2],
      b_so[0::2].reshape(1, _NCOMBO), b_so[1::2].reshape(1, _NCOMBO),
      W_aw, b_aw.reshape(1, _NCOMBO))


def _proj_call(attn2, W_op, b_op):
    grid = (_BS * _NQ // _BQ,)
    return pl.pallas_call(
        _proj_body,
        grid=grid,
        in_specs=[
            pl.BlockSpec((_BQ, _H * _D), lambda i: (i, 0)),
            pl.BlockSpec((_D, _H * _D), lambda i: (0, 0)),
            pl.BlockSpec((1, _D), lambda i: (0, 0)),
        ],
        out_specs=pl.BlockSpec((_BQ, _D), lambda i: (i, 0)),
        out_shape=jax.ShapeDtypeStruct((_BS * _NQ, _D), jnp.float32),
    )(attn2, W_op, b_op.reshape(1, _D))


def kernel(query, value, query_location, spatial_shapes, level_start_index,
           W_so, b_so, W_aw, b_aw, W_op, b_op):
    idx, wt = _prep_call(query, query_location, W_so, b_so, W_aw, b_aw)
    value_flat = value.reshape(_BS * _NV, _D)
    attn2 = _sc_attend(value_flat, idx.reshape(-1), wt.reshape(-1))
    out = _proj_call(attn2, W_op, b_op)
    return out.reshape(_BS, _NQ, _D)
